# Initial kernel scaffold; baseline (speedup 1.0000x reference)
import jax
import jax.numpy as jnp
from jax.experimental import pallas as pl


def kernel(img):
    raise NotImplementedError("write your pallas kernel here")


# speedup vs baseline: 828.0919x; 828.0919x over previous
import functools

import jax
import jax.numpy as jnp
from jax.experimental import pallas as pl
from jax.experimental.pallas import tpu as pltpu

STRONG = 255.0
WEAK = 25.0
LOW_T = 0.05
HIGH_T = 0.15

H = W = 512


def _sr(a, s):
    z = jnp.zeros(a.shape[:-1] + (s,), a.dtype)
    return jnp.concatenate([z, a[..., : a.shape[-1] - s]], axis=-1)


def _sl(a, s):
    z = jnp.zeros(a.shape[:-1] + (s,), a.dtype)
    return jnp.concatenate([a[..., s:], z], axis=-1)


def _sd(a, s):
    z = jnp.zeros((s,) + a.shape[1:], a.dtype)
    return jnp.concatenate([z, a[: a.shape[0] - s]], axis=0)


def _su(a, s):
    z = jnp.zeros((s,) + a.shape[1:], a.dtype)
    return jnp.concatenate([a[s:], z], axis=0)


def _hyst_kernel(x_ref, o_ref, w_ref, d_ref, p_ref):
    x = x_ref[...]
    hi = jnp.max(x) * HIGH_T
    lo = hi * LOW_T
    strong = (x > hi).astype(jnp.float32)
    weakb = jnp.logical_and(x >= lo, x <= hi)
    weak = weakb.astype(jnp.float32)

    h3 = jnp.maximum(strong, jnp.maximum(_sr(strong, 1), _sl(strong, 1)))
    d = jnp.maximum(
        jnp.maximum(_sr(strong, 1), _sl(strong, 1)),
        jnp.maximum(_sd(h3, 1), _su(h3, 1)),
    )

    col = jax.lax.broadcasted_iota(jnp.int32, (H, W), 1)
    row = jax.lax.broadcasted_iota(jnp.int32, (H, W), 0)
    incol = jnp.logical_and(col > 0, col < W - 1)
    w_ref[...] = weak * incol.astype(jnp.float32)
    d_ref[...] = d
    p_ref[...] = jnp.zeros_like(x)

    def row_body(i, p_prev):
        c = jnp.maximum(p_prev, jnp.maximum(_sr(p_prev, 1), _sl(p_prev, 1)))
        wi = w_ref[pl.ds(i, 1), :]
        a = jnp.maximum(d_ref[pl.ds(i, 1), :], c)
        g = wi * a
        t = wi
        for s in (1, 2, 4, 8, 16, 32, 64, 128, 256):
            g = jnp.maximum(g, t * _sr(g, s))
            t = t * _sr(t, s)
        p_ref[pl.ds(i, 1), :] = g
        return g

    jax.lax.fori_loop(1, H - 1, row_body, jnp.zeros((1, W), jnp.float32))

    p = p_ref[...]
    tx = jnp.where(weakb, WEAK, jnp.where(x >= hi, STRONG, 0.0))
    interior = jnp.logical_and(jnp.logical_and(row > 0, row < H - 1), incol)
    o_ref[...] = jnp.where(
        jnp.logical_and(weakb, interior),
        jnp.where(p > 0.5, STRONG, 0.0),
        tx,
    )


@functools.partial(jax.jit)
def kernel(img):
    x = img.reshape(H, W)
    out = pl.pallas_call(
        _hyst_kernel,
        out_shape=jax.ShapeDtypeStruct((H, W), jnp.float32),
        scratch_shapes=[
            pltpu.VMEM((H, W), jnp.float32),
            pltpu.VMEM((H, W), jnp.float32),
            pltpu.VMEM((H, W), jnp.float32),
        ],
    )(x)
    return out[None, None, :, :]
